# Initial kernel scaffold; baseline (speedup 1.0000x reference)
import jax
import jax.numpy as jnp
from jax.experimental import pallas as pl


def kernel(x, embed):
    raise NotImplementedError("write your pallas kernel here")


# speedup vs baseline: 1.5765x; 1.5765x over previous
import functools

import jax
import jax.numpy as jnp
from jax import lax
from jax.experimental import pallas as pl
from jax.experimental.pallas import tpu as pltpu
from jax.experimental.pallas import tpu_sc as plsc

_BN = 512


def _argmin_body(x_ref, e_ref, o_ref):
    x = x_ref[...]
    e = e_ref[...]
    dot = lax.dot_general(x, e, (((1,), (1,)), ((), ())),
                          preferred_element_type=jnp.float32)
    xx = jnp.sum(x * x, axis=1, keepdims=True)
    ee = jnp.sum(e * e, axis=1)
    dist = -(xx - 2.0 * dot + ee[None, :])
    o_ref[0, 0, :] = jnp.argmax(dist, axis=1).astype(jnp.int32)


def _argmin_indices(xf, embed):
    n, d = xf.shape
    k = embed.shape[0]
    nb = n // _BN
    out = pl.pallas_call(
        _argmin_body,
        grid=(nb,),
        in_specs=[
            pl.BlockSpec((_BN, d), lambda i: (i, 0)),
            pl.BlockSpec((k, d), lambda i: (0, 0)),
        ],
        out_specs=pl.BlockSpec((1, 1, _BN), lambda i: (i, 0, 0)),
        out_shape=jax.ShapeDtypeStruct((nb, 1, _BN), jnp.int32),
    )(xf, embed)
    return out.reshape(n)


def _gather_rows(table, idx):
    info = plsc.get_sparse_core_info()
    nc, ns = info.num_cores, info.num_subcores
    nw = nc * ns
    n = idx.shape[0]
    d = table.shape[1]
    b_per_w = n // nw
    ch = 192
    nch = b_per_w // ch
    mesh = plsc.VectorSubcoreMesh(core_axis_name="c", subcore_axis_name="s")

    @functools.partial(
        pl.kernel, mesh=mesh,
        out_type=jax.ShapeDtypeStruct((n, d), jnp.float32),
        scratch_types=[
            pltpu.VMEM((ch,), jnp.int32),
            pltpu.VMEM((ch, d), jnp.float32),
            pltpu.SemaphoreType.DMA,
        ],
    )
    def k(table_hbm, idx_hbm, out_hbm, idx_v, rows_v, sem):
        wid = lax.axis_index("s") * nc + lax.axis_index("c")
        base = wid * b_per_w
        for c in range(nch):
            off = base + c * ch
            pltpu.sync_copy(idx_hbm.at[pl.ds(off, ch)], idx_v)
            pltpu.async_copy(table_hbm.at[idx_v], rows_v, sem).wait()
            pltpu.sync_copy(rows_v, out_hbm.at[pl.ds(off, ch)])

    return k(table, idx)


def kernel(x, embed):
    shape = x.shape
    d = shape[-1]
    xf = x.reshape(-1, d)
    idx = _argmin_indices(xf, embed)
    quantize = _gather_rows(embed, idx)
    return (quantize.reshape(shape), idx.reshape(shape[:-1]))
